# baseline (speedup 1.0000x reference)
import functools

import jax
import jax.numpy as jnp
from jax import lax
from jax.experimental import pallas as pl
from jax.experimental.pallas import tpu as pltpu
from jax.experimental.pallas import tpu_sc as plsc

NEG = -3.4e38


def _sc_gather(table, idx):
    N = idx.shape[0]
    R, D = table.shape
    NW = 32
    G = N // (NW * 128)
    assert N == NW * G * 128
    NB = min(4, G)
    mesh = plsc.VectorSubcoreMesh(core_axis_name="c", subcore_axis_name="s")
    idx3 = idx.reshape(NW, G, 128)
    dt = table.dtype

    @functools.partial(
        pl.kernel,
        out_type=jax.ShapeDtypeStruct((N, D), dt),
        mesh=mesh,
        scratch_types=(
            [pltpu.VMEM((G, 128), jnp.int32),
             pltpu.VMEM((NB, 128, D), dt)]
            + [pltpu.SemaphoreType.DMA] * NB
            + [pltpu.SemaphoreType.DMA] * NB
        ),
    )
    def gk(table_hbm, idx_hbm, out_hbm, idx_v, rows_v, *sems):
        gsem = sems[:NB]
        ssem = sems[NB:]
        wid = lax.axis_index("s") * 2 + lax.axis_index("c")
        base = wid * (G * 128)
        pltpu.sync_copy(idx_hbm.at[wid], idx_v)

        def start_gather(g, slot):
            return pltpu.async_copy(
                table_hbm.at[idx_v.at[g]], rows_v.at[slot], gsem[slot])

        descs = [None] * NB
        for g in range(NB):
            descs[g] = start_gather(g, g)
        for g in range(G):
            slot = g % NB
            descs[slot].wait()
            st = pltpu.async_copy(
                rows_v.at[slot], out_hbm.at[pl.ds(base + g * 128, 128)],
                ssem[slot])
            st.wait()
            if g + NB < G:
                descs[slot] = start_gather(g + NB, slot)

    return gk(table, idx3)


def _t1_user_rep(seq_emb, ids_p, W_llm, b_llm, interpret=False):
    B, HP, D = seq_emb.shape
    BB = min(B, 256)

    def body(emb_ref, ids_ref, w_ref, b_ref, out_ref):
        ids = ids_ref[:]
        valid = (ids != 0).astype(jnp.float32)
        cnt = jnp.sum(valid, axis=1, keepdims=True)
        s = jnp.sum(emb_ref[:] * valid[:, :, None], axis=1)
        pooled = s / jnp.maximum(cnt, 1.0)
        out_ref[:] = jnp.tanh(
            lax.dot_general(pooled, w_ref[:], (((1,), (0,)), ((), ())),
                            preferred_element_type=jnp.float32) + b_ref[:])

    return pl.pallas_call(
        body,
        grid=(B // BB,),
        in_specs=[
            pl.BlockSpec((BB, HP, D), lambda i: (i, 0, 0)),
            pl.BlockSpec((BB, HP), lambda i: (i, 0)),
            pl.BlockSpec((D, D), lambda i: (0, 0)),
            pl.BlockSpec((1, D), lambda i: (0, 0)),
        ],
        out_specs=pl.BlockSpec((BB, D), lambda i: (i, 0)),
        out_shape=jax.ShapeDtypeStruct((B, D), jnp.float32),
        interpret=interpret,
    )(seq_emb, ids_p, W_llm, b_llm)


def _t2_scores(user_rep, E, tile, interpret=False):
    B, D = user_rep.shape
    V = E.shape[0]
    NT = -(-V // tile)
    WS = NT * tile
    NCT = tile // 128

    def body(u_ref, e_ref, s_ref, cm_ref):
        t = pl.program_id(0)
        S = lax.dot_general(u_ref[:], e_ref[:], (((1,), (1,)), ((), ())),
                            preferred_element_type=jnp.float32)
        col = t * tile + lax.broadcasted_iota(jnp.int32, (B, tile), 1)
        Sm = jnp.where(col < V, S, NEG)
        s_ref[:] = Sm
        cm_ref[0] = jnp.max(Sm.reshape(B, NCT, 128), axis=2)

    return pl.pallas_call(
        body,
        grid=(NT,),
        in_specs=[
            pl.BlockSpec((B, D), lambda t: (0, 0)),
            pl.BlockSpec((tile, D), lambda t: (t, 0)),
        ],
        out_specs=[
            pl.BlockSpec((B, tile), lambda t: (0, t)),
            pl.BlockSpec((1, B, NCT), lambda t: (t, 0, 0)),
        ],
        out_shape=[
            jax.ShapeDtypeStruct((B, WS), jnp.float32),
            jax.ShapeDtypeStruct((NT, B, NCT), jnp.float32),
        ],
        interpret=interpret,
    )(user_rep, E)


def _t3_top_chunks(cm, K, interpret=False):
    B, NCH = cm.shape

    def body(cm_ref, out_ref):
        x = cm_ref[:]
        iota_c = lax.broadcasted_iota(jnp.int32, (B, NCH), 1)
        iota_k = lax.broadcasted_iota(jnp.int32, (B, K), 1)
        out = jnp.zeros((B, K), jnp.int32)
        for k in range(K):
            m = jnp.max(x, axis=1, keepdims=True)
            idx = jnp.min(jnp.where(x == m, iota_c, jnp.int32(2**30)),
                          axis=1, keepdims=True)
            out = out + jnp.where(iota_k == k, idx, 0)
            x = jnp.where(iota_c == idx, NEG, x)
        out_ref[:] = out

    return pl.pallas_call(
        body,
        grid=(1,),
        in_specs=[pl.BlockSpec((B, NCH), lambda i: (0, 0))],
        out_specs=pl.BlockSpec((B, K), lambda i: (0, 0)),
        out_shape=jax.ShapeDtypeStruct((B, K), jnp.int32),
        interpret=interpret,
    )(cm)


def _t6_topk_items(cand, chunk_ids, V, K, interpret=False):
    B, NC = cand.shape
    KCH = chunk_ids.shape[1]
    BB = min(B, 256)

    def body(c_ref, ch_ref, out_ref):
        ch = ch_ref[:]
        g3 = ch[:, :, None] * 128 + lax.broadcasted_iota(
            jnp.int32, (BB, KCH, 128), 2)
        gidx = g3.reshape(BB, NC)
        x = jnp.where(gidx < V, c_ref[:], NEG)
        iota_k = lax.broadcasted_iota(jnp.int32, (BB, K), 1)
        out = jnp.zeros((BB, K), jnp.int32)
        for k in range(K):
            m = jnp.max(x, axis=1, keepdims=True)
            item = jnp.min(jnp.where(x == m, gidx, jnp.int32(2**30)),
                           axis=1, keepdims=True)
            out = out + jnp.where(iota_k == k, item, 0)
            x = jnp.where(gidx == item, NEG, x)
        out_ref[:] = out

    return pl.pallas_call(
        body,
        grid=(B // BB,),
        in_specs=[
            pl.BlockSpec((BB, NC), lambda i: (i, 0)),
            pl.BlockSpec((BB, KCH), lambda i: (i, 0)),
        ],
        out_specs=pl.BlockSpec((BB, K), lambda i: (i, 0)),
        out_shape=jax.ShapeDtypeStruct((B, K), jnp.int32),
        interpret=interpret,
    )(cand, chunk_ids)


def _t7a_fuse(user_rep, retr_rows, W_fusion, b_fusion, ln_gamma,
              ln_beta, K, interpret=False):
    BK, D = retr_rows.shape
    B = user_rep.shape[0]
    BB = min(B, 256)

    def body(u_ref, r_ref, wf_ref, bf_ref, lg_ref, lb_ref, out_ref):
        u = u_ref[:]
        retr = jnp.sum(r_ref[:].reshape(BB, K, D), axis=1) / float(K)
        wf = wf_ref[:]
        gate_in = (
            lax.dot_general(u, wf[:D], (((1,), (0,)), ((), ())),
                            preferred_element_type=jnp.float32)
            + lax.dot_general(retr, wf[D:], (((1,), (0,)), ((), ())),
                              preferred_element_type=jnp.float32)
            + bf_ref[:])
        g = jax.nn.sigmoid(gate_in)
        fused = g * u + (1.0 - g) * retr
        mu = jnp.mean(fused, axis=1, keepdims=True)
        var = jnp.mean((fused - mu) ** 2, axis=1, keepdims=True)
        out_ref[:] = ((fused - mu) / jnp.sqrt(var + 1e-5) * lg_ref[:]
                      + lb_ref[:])

    return pl.pallas_call(
        body,
        grid=(B // BB,),
        in_specs=[
            pl.BlockSpec((BB, D), lambda i: (i, 0)),
            pl.BlockSpec((BB * K, D), lambda i: (i, 0)),
            pl.BlockSpec((2 * D, D), lambda i: (0, 0)),
            pl.BlockSpec((1, D), lambda i: (0, 0)),
            pl.BlockSpec((1, D), lambda i: (0, 0)),
            pl.BlockSpec((1, D), lambda i: (0, 0)),
        ],
        out_specs=pl.BlockSpec((BB, D), lambda i: (i, 0)),
        out_shape=jax.ShapeDtypeStruct((B, D), jnp.float32),
        interpret=interpret,
    )(user_rep, retr_rows, W_fusion, b_fusion, ln_gamma, ln_beta)


def _t7b_logits(fused, W_proj, b_proj, tile, interpret=False):
    B, D = fused.shape
    V = W_proj.shape[1]
    NT = -(-V // tile)

    def body(f_ref, w_ref, b_ref, out_ref):
        out_ref[:] = lax.dot_general(
            f_ref[:], w_ref[:], (((1,), (0,)), ((), ())),
            preferred_element_type=jnp.float32) + b_ref[:]

    return pl.pallas_call(
        body,
        grid=(NT,),
        in_specs=[
            pl.BlockSpec((B, D), lambda t: (0, 0)),
            pl.BlockSpec((D, tile), lambda t: (0, t)),
            pl.BlockSpec((1, tile), lambda t: (0, t)),
        ],
        out_specs=pl.BlockSpec((B, tile), lambda t: (0, t)),
        out_shape=jax.ShapeDtypeStruct((B, V), jnp.float32),
        interpret=interpret,
    )(fused, W_proj, b_proj)


def kernel(sequence_ids, item_embeddings, W_llm, b_llm, W_fusion, b_fusion,
           ln_gamma, ln_beta, W_proj, b_proj):
    B, H = sequence_ids.shape
    V, D = item_embeddings.shape
    K = 20
    TILE = 4096

    ids = sequence_ids.astype(jnp.int32)
    HP = -(-H // 8) * 8
    while (B * HP) % (32 * 128) != 0:
        HP += 8
    ids_p = jnp.concatenate(
        [ids, jnp.zeros((B, HP - H), jnp.int32)], axis=1)
    flat = ids_p.reshape(B * HP)
    spread = jnp.arange(B * HP, dtype=jnp.int32) % V
    idx1 = jnp.where(flat == 0, spread, flat - 1)

    seq_emb = _sc_gather(item_embeddings, idx1).reshape(B, HP, D)
    user_rep = _t1_user_rep(seq_emb, ids_p, W_llm, b_llm.reshape(1, D))

    scores, cm3 = _t2_scores(user_rep, item_embeddings, TILE)
    NCH = cm3.shape[0] * cm3.shape[2]
    cm = cm3.transpose(1, 0, 2).reshape(B, NCH)

    chunk_ids = _t3_top_chunks(cm, K)
    flat_rows = (chunk_ids
                 + NCH * jnp.arange(B, dtype=jnp.int32)[:, None]).reshape(B * K)
    cand = _sc_gather(scores.reshape(B * NCH, 128), flat_rows).reshape(
        B, K * 128)

    topk = _t6_topk_items(cand, chunk_ids, V, K)
    retr_rows = _sc_gather(item_embeddings, topk.reshape(B * K))

    fused = _t7a_fuse(user_rep, retr_rows, W_fusion, b_fusion.reshape(1, D),
                      ln_gamma.reshape(1, D), ln_beta.reshape(1, D), K)
    return _t7b_logits(fused, W_proj, b_proj.reshape(1, V), TILE)


# speedup vs baseline: 1.3396x; 1.3396x over previous
import functools

import jax
import jax.numpy as jnp
from jax import lax
from jax.experimental import pallas as pl
from jax.experimental.pallas import tpu as pltpu
from jax.experimental.pallas import tpu_sc as plsc

NEG = -3.4e38


def _sc_gather(table, idx):
    N = idx.shape[0]
    R, D = table.shape
    NW = 32
    G = N // (NW * 128)
    assert N == NW * G * 128
    NB = min(4, G)
    mesh = plsc.VectorSubcoreMesh(core_axis_name="c", subcore_axis_name="s")
    idx3 = idx.reshape(NW, G, 128)
    dt = table.dtype

    @functools.partial(
        pl.kernel,
        out_type=jax.ShapeDtypeStruct((N, D), dt),
        mesh=mesh,
        scratch_types=(
            [pltpu.VMEM((G, 128), jnp.int32),
             pltpu.VMEM((NB, 128, D), dt)]
            + [pltpu.SemaphoreType.DMA] * NB
            + [pltpu.SemaphoreType.DMA] * NB
        ),
    )
    def gk(table_hbm, idx_hbm, out_hbm, idx_v, rows_v, *sems):
        gsem = sems[:NB]
        ssem = sems[NB:]
        wid = lax.axis_index("s") * 2 + lax.axis_index("c")
        base = wid * (G * 128)
        pltpu.sync_copy(idx_hbm.at[wid], idx_v)

        def start_gather(g, slot):
            return pltpu.async_copy(
                table_hbm.at[idx_v.at[g]], rows_v.at[slot], gsem[slot])

        descs = [None] * NB
        for g in range(NB):
            descs[g] = start_gather(g, g)
        for g in range(G):
            slot = g % NB
            descs[slot].wait()
            st = pltpu.async_copy(
                rows_v.at[slot], out_hbm.at[pl.ds(base + g * 128, 128)],
                ssem[slot])
            st.wait()
            if g + NB < G:
                descs[slot] = start_gather(g + NB, slot)

    return gk(table, idx3)


def _t1_user_rep(seq_emb, ids_p, W_llm, b_llm, interpret=False):
    B, HP, D = seq_emb.shape
    BB = min(B, 256)

    def body(emb_ref, ids_ref, w_ref, b_ref, out_ref):
        ids = ids_ref[:]
        valid = (ids != 0).astype(jnp.float32)
        cnt = jnp.sum(valid, axis=1, keepdims=True)
        s = jnp.sum(emb_ref[:] * valid[:, :, None], axis=1)
        pooled = s / jnp.maximum(cnt, 1.0)
        out_ref[:] = jnp.tanh(
            lax.dot_general(pooled, w_ref[:], (((1,), (0,)), ((), ())),
                            preferred_element_type=jnp.float32) + b_ref[:])

    return pl.pallas_call(
        body,
        grid=(B // BB,),
        in_specs=[
            pl.BlockSpec((BB, HP, D), lambda i: (i, 0, 0)),
            pl.BlockSpec((BB, HP), lambda i: (i, 0)),
            pl.BlockSpec((D, D), lambda i: (0, 0)),
            pl.BlockSpec((1, D), lambda i: (0, 0)),
        ],
        out_specs=pl.BlockSpec((BB, D), lambda i: (i, 0)),
        out_shape=jax.ShapeDtypeStruct((B, D), jnp.float32),
        interpret=interpret,
    )(seq_emb, ids_p, W_llm, b_llm)


def _t2_scores(user_rep, E, tile, interpret=False):
    B, D = user_rep.shape
    V = E.shape[0]
    NT = -(-V // tile)
    WS = NT * tile
    NCT = tile // 128

    def body(u_ref, e_ref, s_ref, cm_ref):
        t = pl.program_id(0)
        S = lax.dot_general(u_ref[:], e_ref[:], (((1,), (1,)), ((), ())),
                            preferred_element_type=jnp.float32)
        col = t * tile + lax.broadcasted_iota(jnp.int32, (B, tile), 1)
        Sm = jnp.where(col < V, S, NEG).reshape(B, NCT, 128)
        s_ref[:] = Sm
        cm_ref[0] = jnp.max(Sm, axis=2)

    return pl.pallas_call(
        body,
        grid=(NT,),
        in_specs=[
            pl.BlockSpec((B, D), lambda t: (0, 0)),
            pl.BlockSpec((tile, D), lambda t: (t, 0)),
        ],
        out_specs=[
            pl.BlockSpec((B, NCT, 128), lambda t: (0, t, 0)),
            pl.BlockSpec((1, B, NCT), lambda t: (t, 0, 0)),
        ],
        out_shape=[
            jax.ShapeDtypeStruct((B, WS // 128, 128), jnp.float32),
            jax.ShapeDtypeStruct((NT, B, NCT), jnp.float32),
        ],
        interpret=interpret,
    )(user_rep, E)


def _t3_top_chunks(cm, K, interpret=False):
    B, NCH = cm.shape

    def body(cm_ref, out_ref):
        x = cm_ref[:]
        iota_c = lax.broadcasted_iota(jnp.int32, (B, NCH), 1)
        iota_k = lax.broadcasted_iota(jnp.int32, (B, K), 1)
        out = jnp.zeros((B, K), jnp.int32)
        for k in range(K):
            m = jnp.max(x, axis=1, keepdims=True)
            idx = jnp.min(jnp.where(x == m, iota_c, jnp.int32(2**30)),
                          axis=1, keepdims=True)
            out = out + jnp.where(iota_k == k, idx, 0)
            x = jnp.where(iota_c == idx, NEG, x)
        out_ref[:] = out

    return pl.pallas_call(
        body,
        grid=(1,),
        in_specs=[pl.BlockSpec((B, NCH), lambda i: (0, 0))],
        out_specs=pl.BlockSpec((B, K), lambda i: (0, 0)),
        out_shape=jax.ShapeDtypeStruct((B, K), jnp.int32),
        interpret=interpret,
    )(cm)


def _t6_topk_items(cand, chunk_ids, V, K, interpret=False):
    B, NC = cand.shape
    KCH = chunk_ids.shape[1]
    BB = min(B, 256)

    def body(c_ref, ch_ref, out_ref):
        ch = ch_ref[:]
        g3 = ch[:, :, None] * 128 + lax.broadcasted_iota(
            jnp.int32, (BB, KCH, 128), 2)
        gidx = g3.reshape(BB, NC)
        x = jnp.where(gidx < V, c_ref[:], NEG)
        iota_k = lax.broadcasted_iota(jnp.int32, (BB, K), 1)
        out = jnp.zeros((BB, K), jnp.int32)
        for k in range(K):
            m = jnp.max(x, axis=1, keepdims=True)
            item = jnp.min(jnp.where(x == m, gidx, jnp.int32(2**30)),
                           axis=1, keepdims=True)
            out = out + jnp.where(iota_k == k, item, 0)
            x = jnp.where(gidx == item, NEG, x)
        out_ref[:] = out

    return pl.pallas_call(
        body,
        grid=(B // BB,),
        in_specs=[
            pl.BlockSpec((BB, NC), lambda i: (i, 0)),
            pl.BlockSpec((BB, KCH), lambda i: (i, 0)),
        ],
        out_specs=pl.BlockSpec((BB, K), lambda i: (i, 0)),
        out_shape=jax.ShapeDtypeStruct((B, K), jnp.int32),
        interpret=interpret,
    )(cand, chunk_ids)


def _t7a_fuse(user_rep, retr_rows, W_fusion, b_fusion, ln_gamma,
              ln_beta, K, interpret=False):
    BK, D = retr_rows.shape
    B = user_rep.shape[0]
    BB = min(B, 256)

    def body(u_ref, r_ref, wf_ref, bf_ref, lg_ref, lb_ref, out_ref):
        u = u_ref[:]
        retr = jnp.sum(r_ref[:].reshape(BB, K, D), axis=1) / float(K)
        wf = wf_ref[:]
        gate_in = (
            lax.dot_general(u, wf[:D], (((1,), (0,)), ((), ())),
                            preferred_element_type=jnp.float32)
            + lax.dot_general(retr, wf[D:], (((1,), (0,)), ((), ())),
                              preferred_element_type=jnp.float32)
            + bf_ref[:])
        g = jax.nn.sigmoid(gate_in)
        fused = g * u + (1.0 - g) * retr
        mu = jnp.mean(fused, axis=1, keepdims=True)
        var = jnp.mean((fused - mu) ** 2, axis=1, keepdims=True)
        out_ref[:] = ((fused - mu) / jnp.sqrt(var + 1e-5) * lg_ref[:]
                      + lb_ref[:])

    return pl.pallas_call(
        body,
        grid=(B // BB,),
        in_specs=[
            pl.BlockSpec((BB, D), lambda i: (i, 0)),
            pl.BlockSpec((BB * K, D), lambda i: (i, 0)),
            pl.BlockSpec((2 * D, D), lambda i: (0, 0)),
            pl.BlockSpec((1, D), lambda i: (0, 0)),
            pl.BlockSpec((1, D), lambda i: (0, 0)),
            pl.BlockSpec((1, D), lambda i: (0, 0)),
        ],
        out_specs=pl.BlockSpec((BB, D), lambda i: (i, 0)),
        out_shape=jax.ShapeDtypeStruct((B, D), jnp.float32),
        interpret=interpret,
    )(user_rep, retr_rows, W_fusion, b_fusion, ln_gamma, ln_beta)


def _t7b_logits(fused, W_proj, b_proj, tile, interpret=False):
    B, D = fused.shape
    V = W_proj.shape[1]
    NT = -(-V // tile)

    def body(f_ref, w_ref, b_ref, out_ref):
        out_ref[:] = lax.dot_general(
            f_ref[:], w_ref[:], (((1,), (0,)), ((), ())),
            preferred_element_type=jnp.float32) + b_ref[:]

    return pl.pallas_call(
        body,
        grid=(NT,),
        in_specs=[
            pl.BlockSpec((B, D), lambda t: (0, 0)),
            pl.BlockSpec((D, tile), lambda t: (0, t)),
            pl.BlockSpec((1, tile), lambda t: (0, t)),
        ],
        out_specs=pl.BlockSpec((B, tile), lambda t: (0, t)),
        out_shape=jax.ShapeDtypeStruct((B, V), jnp.float32),
        interpret=interpret,
    )(fused, W_proj, b_proj)


def kernel(sequence_ids, item_embeddings, W_llm, b_llm, W_fusion, b_fusion,
           ln_gamma, ln_beta, W_proj, b_proj):
    B, H = sequence_ids.shape
    V, D = item_embeddings.shape
    K = 20
    TILE = 4096

    ids = sequence_ids.astype(jnp.int32)
    HP = -(-H // 8) * 8
    while (B * HP) % (32 * 128) != 0:
        HP += 8
    ids_p = jnp.concatenate(
        [ids, jnp.zeros((B, HP - H), jnp.int32)], axis=1)
    flat = ids_p.reshape(B * HP)
    spread = jnp.arange(B * HP, dtype=jnp.int32) % V
    idx1 = jnp.where(flat == 0, spread, flat - 1)

    seq_emb = _sc_gather(item_embeddings, idx1).reshape(B, HP, D)
    user_rep = _t1_user_rep(seq_emb, ids_p, W_llm, b_llm.reshape(1, D))

    scores, cm3 = _t2_scores(user_rep, item_embeddings, TILE)
    NCH = cm3.shape[0] * cm3.shape[2]
    cm = cm3.transpose(1, 0, 2).reshape(B, NCH)

    chunk_ids = _t3_top_chunks(cm, K)
    flat_rows = (chunk_ids
                 + NCH * jnp.arange(B, dtype=jnp.int32)[:, None]).reshape(B * K)
    cand = _sc_gather(scores.reshape(B * NCH, 128), flat_rows).reshape(
        B, K * 128)

    topk = _t6_topk_items(cand, chunk_ids, V, K)
    retr_rows = _sc_gather(item_embeddings, topk.reshape(B * K))

    fused = _t7a_fuse(user_rep, retr_rows, W_fusion, b_fusion.reshape(1, D),
                      ln_gamma.reshape(1, D), ln_beta.reshape(1, D), K)
    return _t7b_logits(fused, W_proj, b_proj.reshape(1, V), TILE)
